# baseline (speedup 1.0000x reference)
import jax
import jax.numpy as jnp
from jax.experimental import pallas as pl


N_NODES = 100000
N_NEIGH = 64
BR = 2048


def _body(x_ref, w_ref, b_ref, o_ref):
    o_ref[...] = b_ref[...] + jnp.sum(x_ref[...] * w_ref[...], axis=1)


def kernel(layer_input, weight, bias):
    grid = (pl.cdiv(N_NODES, BR),)
    return pl.pallas_call(
        _body,
        grid=grid,
        in_specs=[
            pl.BlockSpec((BR, N_NEIGH), lambda i: (i, 0)),
            pl.BlockSpec((BR, N_NEIGH), lambda i: (i, 0)),
            pl.BlockSpec((BR,), lambda i: (i,)),
        ],
        out_specs=pl.BlockSpec((BR,), lambda i: (i,)),
        out_shape=jax.ShapeDtypeStruct((N_NODES,), jnp.float32),
    )(layer_input, weight, bias)
